# baseline (speedup 1.0000x reference)
import functools

import jax
import jax.numpy as jnp
from jax import lax
from jax.experimental import pallas as pl
from jax.experimental.pallas import tpu as pltpu
from jax.experimental.pallas import tpu_sc as plsc

BATCH = 16384
HIDDEN = 64
GAMMA = 12.0

NUM_CORES = 2
NUM_SUBCORES = 16
NW = NUM_CORES * NUM_SUBCORES
B_PER_W = BATCH // NW
CHUNK = 128
N_CHUNKS = B_PER_W // CHUNK
LANES = 16
UNROLL = 8


def _sc_body(heads_hbm, rels_hbm, tails_hbm, ent_hbm, rel_hbm, out_hbm,
             idx_h, idx_r, idx_t, rows_h, rows_r, rows_t, out_v, sem):
    wid = lax.axis_index("s") * NUM_CORES + lax.axis_index("c")
    crow = wid * N_CHUNKS

    pltpu.sync_copy(heads_hbm.at[pl.ds(crow, N_CHUNKS)], idx_h)
    pltpu.sync_copy(rels_hbm.at[pl.ds(crow, N_CHUNKS)], idx_r)
    pltpu.sync_copy(tails_hbm.at[pl.ds(crow, N_CHUNKS)], idx_t)

    copies = []
    for j in range(N_CHUNKS):
        dst = pl.ds(j * CHUNK, CHUNK)
        copies.append(pltpu.async_copy(ent_hbm.at[idx_h.at[j]], rows_h.at[dst], sem))
        copies.append(pltpu.async_copy(rel_hbm.at[idx_r.at[j]], rows_r.at[dst], sem))
        copies.append(pltpu.async_copy(ent_hbm.at[idx_t.at[j]], rows_t.at[dst], sem))
    for c in copies:
        c.wait()

    lane = lax.iota(jnp.int32, LANES)
    dnums = lax.GatherDimensionNumbers(
        offset_dims=(), collapsed_slice_dims=(0,), start_index_map=(0,))

    def _shuffle(x, idx):
        return lax.gather(x, idx[:, None], dnums, slice_sizes=(1,),
                          mode=lax.GatherScatterMode.PROMISE_IN_BOUNDS)

    def row_group(g, carry):
        out_vec = jnp.zeros((LANES,), jnp.float32)
        for u in range(LANES):
            i = g * LANES + u
            acc = None
            for k in range(HIDDEN // LANES):
                sl = pl.ds(k * LANES, LANES)
                d = jnp.abs(rows_h[i, sl] + rows_r[i, sl] - rows_t[i, sl])
                acc = d if acc is None else acc + d
            for sh in (8, 4, 2, 1):
                acc = acc + _shuffle(acc, lane ^ sh)
            out_vec = jnp.where(lane == u, GAMMA - acc, out_vec)
        out_v[pl.ds(g * LANES, LANES)] = out_vec
        return carry

    lax.fori_loop(0, B_PER_W // LANES, row_group, 0, unroll=False)

    pltpu.sync_copy(out_v, out_hbm.at[pl.ds(wid * B_PER_W, B_PER_W)])


def _warm_body(out_hbm, v, sem):
    wid = lax.axis_index("s") * NUM_CORES + lax.axis_index("c")

    @pl.when(wid == 0)
    def _():
        v[...] = jnp.zeros((LANES,), jnp.float32)
        pltpu.sync_copy(v, out_hbm)


def _warm():
    mesh = plsc.VectorSubcoreMesh(
        core_axis_name="c", subcore_axis_name="s",
        num_cores=NUM_CORES, num_subcores=NUM_SUBCORES)
    fn = functools.partial(
        pl.kernel,
        out_type=jax.ShapeDtypeStruct((LANES,), jnp.float32),
        mesh=mesh,
        scratch_types=[
            pltpu.VMEM((LANES,), jnp.float32),
            pltpu.SemaphoreType.DMA,
        ],
    )(_warm_body)
    return fn()


@functools.partial(jax.jit, static_argnames=())
def _score(heads, rels, tails, entity_embedding, relation_embedding):
    warm = _warm()
    mesh = plsc.VectorSubcoreMesh(
        core_axis_name="c", subcore_axis_name="s",
        num_cores=NUM_CORES, num_subcores=NUM_SUBCORES)
    fn = functools.partial(
        pl.kernel,
        out_type=jax.ShapeDtypeStruct((BATCH,), jnp.float32),
        mesh=mesh,
        scratch_types=[
            pltpu.VMEM((N_CHUNKS, CHUNK), jnp.int32),
            pltpu.VMEM((N_CHUNKS, CHUNK), jnp.int32),
            pltpu.VMEM((N_CHUNKS, CHUNK), jnp.int32),
            pltpu.VMEM((B_PER_W, HIDDEN), jnp.float32),
            pltpu.VMEM((B_PER_W, HIDDEN), jnp.float32),
            pltpu.VMEM((B_PER_W, HIDDEN), jnp.float32),
            pltpu.VMEM((B_PER_W,), jnp.float32),
            pltpu.SemaphoreType.DMA,
        ],
        compiler_params=pltpu.CompilerParams(use_tc_tiling_on_sc=False),
    )(_sc_body)
    score = fn(heads, rels, tails, entity_embedding, relation_embedding)
    return score + warm[0]


def kernel(sample, entity_embedding, relation_embedding):
    sample = sample.astype(jnp.int32)
    heads = sample[:, 0].reshape(BATCH // CHUNK, CHUNK)
    rels = sample[:, 1].reshape(BATCH // CHUNK, CHUNK)
    tails = sample[:, 2].reshape(BATCH // CHUNK, CHUNK)
    score = _score(heads, rels, tails, entity_embedding, relation_embedding)
    return score.reshape(BATCH, 1)


# speedup vs baseline: 1.1672x; 1.1672x over previous
import functools

import jax
import jax.numpy as jnp
from jax import lax
from jax.experimental import pallas as pl
from jax.experimental.pallas import tpu as pltpu
from jax.experimental.pallas import tpu_sc as plsc

BATCH = 16384
HIDDEN = 64
GAMMA = 12.0

NUM_CORES = 2
NUM_SUBCORES = 16
NW = NUM_CORES * NUM_SUBCORES
B_PER_W = BATCH // NW
CHUNK = 128
N_CHUNKS = B_PER_W // CHUNK
LANES = 16

TBLK = 512
TNBLK = BATCH // TBLK


NQ = 8
QROWS = TBLK // NQ


def _tc_body(idx_r, rel_hbm, out_ref, buf, sems):
    k = pl.program_id(0)

    def issue_block(blk, par):
        def enqueue(r, carry):
            for q in range(NQ):
                ir = idx_r[blk * TBLK + q * QROWS + r]
                pltpu.async_copy(rel_hbm.at[ir],
                                 buf.at[par, q * QROWS + r], sems.at[par])
            return carry

        lax.fori_loop(0, QROWS, enqueue, 0, unroll=2)

    par = lax.rem(k, 2)
    nxt = lax.rem(k + 1, 2)

    @pl.when(k == 0)
    def _():
        issue_block(0, 0)

    @pl.when(k + 1 < TNBLK)
    def _():
        issue_block(k + 1, nxt)

    pltpu.make_async_copy(
        rel_hbm.at[pl.ds(0, TBLK)], buf.at[par], sems.at[par]).wait()
    out_ref[...] = buf[par]


def _sc_body(heads_hbm, tails_hbm, ent_hbm, relrows_hbm, out_hbm,
             idx_h, idx_t, rows_h, rows_t, rows_r, out_v, sem):
    wid = lax.axis_index("s") * NUM_CORES + lax.axis_index("c")
    base = wid * B_PER_W
    crow = wid * N_CHUNKS

    pltpu.sync_copy(heads_hbm.at[pl.ds(crow, N_CHUNKS)], idx_h)
    pltpu.sync_copy(tails_hbm.at[pl.ds(crow, N_CHUNKS)], idx_t)

    copies = [pltpu.async_copy(
        relrows_hbm.at[pl.ds(base, B_PER_W)], rows_r, sem)]
    for j in range(N_CHUNKS):
        dst = pl.ds(j * CHUNK, CHUNK)
        copies.append(pltpu.async_copy(
            ent_hbm.at[idx_h.at[j]], rows_h.at[dst], sem))
        copies.append(pltpu.async_copy(
            ent_hbm.at[idx_t.at[j]], rows_t.at[dst], sem))
    for c in copies:
        c.wait()

    lane = lax.iota(jnp.int32, LANES)
    dnums = lax.GatherDimensionNumbers(
        offset_dims=(), collapsed_slice_dims=(0,), start_index_map=(0,))

    def _shuffle(x, idx):
        return lax.gather(x, idx[:, None], dnums, slice_sizes=(1,),
                          mode=lax.GatherScatterMode.PROMISE_IN_BOUNDS)

    def row_group(g, carry):
        out_vec = jnp.zeros((LANES,), jnp.float32)
        for u in range(LANES):
            i = g * LANES + u
            acc = None
            for k in range(HIDDEN // LANES):
                sl = pl.ds(k * LANES, LANES)
                d = jnp.abs(rows_h[i, sl] + rows_r[i, sl] - rows_t[i, sl])
                acc = d if acc is None else acc + d
            for sh in (8, 4, 2, 1):
                acc = acc + _shuffle(acc, lane ^ sh)
            out_vec = jnp.where(lane == u, GAMMA - acc, out_vec)
        out_v[pl.ds(g * LANES, LANES)] = out_vec
        return carry

    lax.fori_loop(0, B_PER_W // LANES, row_group, 0, unroll=False)

    pltpu.sync_copy(out_v, out_hbm.at[pl.ds(base, B_PER_W)])


@jax.jit
def _score(heads, tails, rels, entity_embedding, relation_embedding):
    grid_spec = pltpu.PrefetchScalarGridSpec(
        num_scalar_prefetch=1,
        grid=(TNBLK,),
        in_specs=[pl.BlockSpec(memory_space=pl.ANY)],
        out_specs=pl.BlockSpec((TBLK, HIDDEN), lambda k, *p: (k, 0)),
        scratch_shapes=[
            pltpu.VMEM((2, TBLK, HIDDEN), jnp.float32),
            pltpu.SemaphoreType.DMA((2,)),
        ],
    )
    rel_rows = pl.pallas_call(
        _tc_body,
        grid_spec=grid_spec,
        out_shape=jax.ShapeDtypeStruct((BATCH, HIDDEN), jnp.float32),
        compiler_params=pltpu.CompilerParams(
            dimension_semantics=("arbitrary",)),
    )(rels, relation_embedding)

    mesh = plsc.VectorSubcoreMesh(
        core_axis_name="c", subcore_axis_name="s",
        num_cores=NUM_CORES, num_subcores=NUM_SUBCORES)
    fn = functools.partial(
        pl.kernel,
        out_type=jax.ShapeDtypeStruct((BATCH,), jnp.float32),
        mesh=mesh,
        scratch_types=[
            pltpu.VMEM((N_CHUNKS, CHUNK), jnp.int32),
            pltpu.VMEM((N_CHUNKS, CHUNK), jnp.int32),
            pltpu.VMEM((B_PER_W, HIDDEN), jnp.float32),
            pltpu.VMEM((B_PER_W, HIDDEN), jnp.float32),
            pltpu.VMEM((B_PER_W, HIDDEN), jnp.float32),
            pltpu.VMEM((B_PER_W,), jnp.float32),
            pltpu.SemaphoreType.DMA,
        ],
        compiler_params=pltpu.CompilerParams(use_tc_tiling_on_sc=False),
    )(_sc_body)
    return fn(heads, tails, entity_embedding, rel_rows)


def kernel(sample, entity_embedding, relation_embedding):
    sample = sample.astype(jnp.int32)
    heads = sample[:, 0].reshape(BATCH // CHUNK, CHUNK)
    tails = sample[:, 2].reshape(BATCH // CHUNK, CHUNK)
    rels = sample[:, 1]
    score = _score(heads, tails, rels, entity_embedding, relation_embedding)
    return score.reshape(BATCH, 1)


# speedup vs baseline: 1.2652x; 1.0840x over previous
import functools

import jax
import jax.numpy as jnp
from jax import lax
from jax.experimental import pallas as pl
from jax.experimental.pallas import tpu as pltpu

BATCH = 16384
HIDDEN = 64
GAMMA = 12.0

BLK = 512
NBLK = BATCH // BLK
NDUP = 2
NQ = 2 * NDUP
QROWS = BLK // NQ


def _body(idx_h, idx_r, idx_t, ent_a, ent_b, rel_a, rel_b, out_ref, *rest):
    bufs = rest[:3 * NQ]
    sems = rest[3 * NQ]
    k = pl.program_id(0)

    idxs = (idx_h, idx_r, idx_t)
    tabs = ((ent_a, ent_b), (rel_a, rel_b), (ent_a, ent_b))

    def issue_block(blk, par):
        def enqueue(r, carry):
            for t in range(3):
                for q in range(NQ):
                    i = idxs[t][blk * BLK + q * QROWS + r]
                    pltpu.async_copy(
                        tabs[t][q % NDUP].at[i],
                        bufs[t * NQ + q].at[par, r],
                        sems.at[par, t * NQ + q])
            return carry

        lax.fori_loop(0, QROWS, enqueue, 0, unroll=2)

    par = lax.rem(k, 2)
    nxt = lax.rem(k + 1, 2)

    @pl.when(k == 0)
    def _():
        issue_block(0, 0)

    @pl.when(k + 1 < NBLK)
    def _():
        issue_block(k + 1, nxt)

    for tq in range(3 * NQ):
        pltpu.make_async_copy(
            ent_a.at[pl.ds(0, QROWS)], bufs[tq].at[par],
            sems.at[par, tq]).wait()

    h = jnp.concatenate([bufs[q][par] for q in range(NQ)], axis=0)
    r = jnp.concatenate([bufs[NQ + q][par] for q in range(NQ)], axis=0)
    t = jnp.concatenate([bufs[2 * NQ + q][par] for q in range(NQ)], axis=0)
    d = jnp.abs(h + r - t)
    out_ref[...] = GAMMA - jnp.sum(d, axis=1, keepdims=True)


@jax.jit
def _score(heads, rels, tails, entity_embedding, relation_embedding):
    grid_spec = pltpu.PrefetchScalarGridSpec(
        num_scalar_prefetch=3,
        grid=(NBLK,),
        in_specs=[pl.BlockSpec(memory_space=pl.ANY)] * 4,
        out_specs=pl.BlockSpec((BLK, 1), lambda k, *p: (k, 0)),
        scratch_shapes=(
            [pltpu.VMEM((2, QROWS, HIDDEN), jnp.float32)
             for _ in range(3 * NQ)]
            + [pltpu.SemaphoreType.DMA((2, 3 * NQ))]),
    )
    fn = pl.pallas_call(
        _body,
        grid_spec=grid_spec,
        out_shape=jax.ShapeDtypeStruct((BATCH, 1), jnp.float32),
        compiler_params=pltpu.CompilerParams(
            dimension_semantics=("arbitrary",)),
    )
    return fn(heads, rels, tails,
              entity_embedding, entity_embedding,
              relation_embedding, relation_embedding)


def kernel(sample, entity_embedding, relation_embedding):
    sample = sample.astype(jnp.int32)
    heads = sample[:, 0]
    rels = sample[:, 1]
    tails = sample[:, 2]
    return _score(heads, rels, tails, entity_embedding, relation_embedding)
